# baseline (speedup 1.0000x reference)
import functools

import jax
import jax.numpy as jnp
from jax import lax
from jax.experimental import pallas as pl
from jax.experimental.pallas import tpu as pltpu
from jax.experimental.pallas import tpu_sc as plsc

NC = 2
NS = 16
NW = NC * NS
CHUNK = 128
GROUP = 4


def _sc_gather(ids_flat, table):
    n = ids_flat.shape[0]
    d = table.shape[1]
    nch = n // (NW * CHUNK)
    ngrp = nch // GROUP
    rows_per_w = nch * CHUNK
    ids3 = ids_flat.reshape(NW, nch, CHUNK)
    mesh = plsc.VectorSubcoreMesh(core_axis_name="c", subcore_axis_name="s")

    @functools.partial(
        pl.kernel,
        out_type=jax.ShapeDtypeStruct((n, d), table.dtype),
        mesh=mesh,
        scratch_types=[
            pltpu.VMEM((nch, CHUNK), jnp.int32),
            pltpu.VMEM((GROUP * CHUNK, d), table.dtype),
            pltpu.SemaphoreType.DMA,
        ],
        compiler_params=pltpu.CompilerParams(use_tc_tiling_on_sc=False),
    )
    def gather_kernel(ids_hbm, table_hbm, out_hbm, idx_v, rows_v, sem):
        wid = lax.axis_index("s") * NC + lax.axis_index("c")
        base = wid * rows_per_w
        pltpu.sync_copy(ids_hbm.at[wid], idx_v)

        @pl.loop(0, ngrp)
        def _grp(g):
            waits = []
            for k in range(GROUP):
                c = g * GROUP + k
                waits.append(pltpu.async_copy(
                    table_hbm.at[idx_v.at[c]],
                    rows_v.at[pl.ds(k * CHUNK, CHUNK)],
                    sem))
            for w in waits:
                w.wait()
            pltpu.sync_copy(
                rows_v,
                out_hbm.at[pl.ds(base + g * (GROUP * CHUNK), GROUP * CHUNK)])

    return gather_kernel(ids3, table)


PERIOD = 512


def _tc_repack(table_t, nb=2048):
    d, v = table_t.shape
    grid = (v + nb - 1) // nb

    def body(in_ref, out_ref):
        out_ref[...] = in_ref[...].T.astype(jnp.bfloat16)

    return pl.pallas_call(
        body,
        grid=(grid,),
        in_specs=[pl.BlockSpec((d, nb), lambda i: (0, i))],
        out_specs=pl.BlockSpec((nb, d), lambda i: (i, 0)),
        out_shape=jax.ShapeDtypeStruct((v, d), jnp.bfloat16),
    )(table_t)


def _tc_encode_t(emb, ego_g, w1, w2, b2, bz):
    n, d = emb.shape
    sl = n // bz
    reps = bz // PERIOD

    def body(emb_ref, ego_ref, w1_ref, w2_ref, b_ref, out_ref):
        base = lax.dot_general(
            w2_ref[...], ego_ref[0],
            (((0,), (0,)), ((), ())),
            preferred_element_type=jnp.float32,
            precision=lax.Precision.HIGHEST) + b_ref[...]
        addend = jnp.concatenate([base] * reps, axis=1)

        h = lax.dot_general(
            w1_ref[...].astype(jnp.bfloat16),
            emb_ref[...].astype(jnp.bfloat16),
            (((0,), (1,)), ((), ())),
            preferred_element_type=jnp.float32)
        out_ref[...] = jnp.maximum(h + addend, 0.0)

    return pl.pallas_call(
        body,
        grid=(sl,),
        in_specs=[
            pl.BlockSpec((bz, d), lambda s: (s, 0)),
            pl.BlockSpec((1, 3, PERIOD), lambda s: (s, 0, 0)),
            pl.BlockSpec(w1.shape, lambda s: (0, 0)),
            pl.BlockSpec(w2.shape, lambda s: (0, 0)),
            pl.BlockSpec(b2.shape, lambda s: (0, 0)),
        ],
        out_specs=pl.BlockSpec((d, bz), lambda s: (s, 0)),
        out_shape=jax.ShapeDtypeStruct((sl * d, bz), jnp.float32),
    )(emb, ego_g, w1, w2, b2)


def kernel(input_ids, ego_info, table, W, b):
    bz, sl = input_ids.shape
    d = table.shape[1]
    ids_flat = input_ids.T.reshape(bz * sl).astype(jnp.int32)
    table_r = _tc_repack(table.T)
    emb = _sc_gather(ids_flat, table_r)
    w1 = W[:d]
    w2 = W[d:]
    b2 = b.reshape(d, 1)
    s_iota = jnp.arange(sl, dtype=jnp.int32)
    m_iota = jnp.arange(PERIOD, dtype=jnp.int32)
    e_idx = (s_iota[:, None] + sl * m_iota[None, :]) % bz
    ego_g = jnp.transpose(ego_info[e_idx], (0, 2, 1))
    out_t = _tc_encode_t(emb, ego_g, w1, w2, b2, bz)
    return out_t.reshape(sl, d, bz).transpose(2, 0, 1)


# speedup vs baseline: 1.6895x; 1.6895x over previous
import functools

import jax
import jax.numpy as jnp
from jax import lax
from jax.experimental import pallas as pl
from jax.experimental.pallas import tpu as pltpu
from jax.experimental.pallas import tpu_sc as plsc

NC = 2
NS = 16
NW = NC * NS
CHUNK = 128
GROUP = 4


def _sc_gather(ids_flat, table):
    n = ids_flat.shape[0]
    d = table.shape[1]
    nch = n // (NW * CHUNK)
    ngrp = nch // GROUP
    rows_per_w = nch * CHUNK
    ids3 = ids_flat.reshape(NW, nch, CHUNK)
    mesh = plsc.VectorSubcoreMesh(core_axis_name="c", subcore_axis_name="s")

    @functools.partial(
        pl.kernel,
        out_type=jax.ShapeDtypeStruct((n, d), table.dtype),
        mesh=mesh,
        scratch_types=[
            pltpu.VMEM((nch, CHUNK), jnp.int32),
            pltpu.VMEM((GROUP * CHUNK, d), table.dtype),
            pltpu.SemaphoreType.DMA,
        ],
        compiler_params=pltpu.CompilerParams(use_tc_tiling_on_sc=False),
    )
    def gather_kernel(ids_hbm, table_hbm, out_hbm, idx_v, rows_v, sem):
        wid = lax.axis_index("s") * NC + lax.axis_index("c")
        base = wid * rows_per_w
        pltpu.sync_copy(ids_hbm.at[wid], idx_v)

        @pl.loop(0, ngrp)
        def _grp(g):
            waits = []
            for k in range(GROUP):
                c = g * GROUP + k
                waits.append(pltpu.async_copy(
                    table_hbm.at[idx_v.at[c]],
                    rows_v.at[pl.ds(k * CHUNK, CHUNK)],
                    sem))
            for w in waits:
                w.wait()
            pltpu.sync_copy(
                rows_v,
                out_hbm.at[pl.ds(base + g * (GROUP * CHUNK), GROUP * CHUNK)])

    return gather_kernel(ids3, table)


PERIOD = 512


def _tc_repack(table_t, nb=2048):
    d, v = table_t.shape
    grid = (v + nb - 1) // nb

    def body(in_ref, out_ref):
        out_ref[...] = in_ref[...].T.astype(jnp.bfloat16)

    return pl.pallas_call(
        body,
        grid=(grid,),
        in_specs=[pl.BlockSpec((d, nb), lambda i: (0, i))],
        out_specs=pl.BlockSpec((nb, d), lambda i: (i, 0)),
        out_shape=jax.ShapeDtypeStruct((v, d), jnp.bfloat16),
    )(table_t)


def _tc_encode_t(emb, ego_g, w1, w2, b2, bz):
    n, d = emb.shape
    sl = n // bz
    reps = bz // PERIOD

    def body(emb_ref, ego_ref, w1_ref, w2_ref, b_ref, out_ref):
        base = lax.dot_general(
            w2_ref[...], ego_ref[0],
            (((0,), (1,)), ((), ())),
            preferred_element_type=jnp.float32,
            precision=lax.Precision.HIGHEST) + b_ref[...]
        addend = jnp.concatenate([base] * reps, axis=1)

        h = lax.dot_general(
            w1_ref[...].astype(jnp.bfloat16),
            emb_ref[...].astype(jnp.bfloat16),
            (((0,), (1,)), ((), ())),
            preferred_element_type=jnp.float32)
        out_ref[...] = jnp.maximum(h + addend, 0.0)

    return pl.pallas_call(
        body,
        grid=(sl,),
        in_specs=[
            pl.BlockSpec((bz, d), lambda s: (s, 0)),
            pl.BlockSpec((1, PERIOD, 3), lambda s: (s, 0, 0)),
            pl.BlockSpec(w1.shape, lambda s: (0, 0)),
            pl.BlockSpec(w2.shape, lambda s: (0, 0)),
            pl.BlockSpec(b2.shape, lambda s: (0, 0)),
        ],
        out_specs=pl.BlockSpec((d, bz), lambda s: (s, 0)),
        out_shape=jax.ShapeDtypeStruct((sl * d, bz), jnp.float32),
    )(emb, ego_g, w1, w2, b2)


def kernel(input_ids, ego_info, table, W, b):
    bz, sl = input_ids.shape
    d = table.shape[1]
    ids_flat = input_ids.T.reshape(bz * sl).astype(jnp.int32)
    emb = _sc_gather(ids_flat, table)
    w1 = W[:d]
    w2 = W[d:]
    b2 = b.reshape(d, 1)
    reps = (PERIOD * sl) // bz
    ego_p = jnp.tile(ego_info, (reps, 1)).reshape(PERIOD, sl, 3)
    ego_p = ego_p.transpose(1, 0, 2)
    out_t = _tc_encode_t(emb, ego_p, w1, w2, b2, bz)
    return out_t.reshape(sl, d, bz).transpose(2, 0, 1)


# speedup vs baseline: 1.9539x; 1.1565x over previous
import functools

import jax
import jax.numpy as jnp
from jax import lax
from jax.experimental import pallas as pl
from jax.experimental.pallas import tpu as pltpu
from jax.experimental.pallas import tpu_sc as plsc

NC = 2
NS = 16
NW = NC * NS
CHUNK = 128
GROUP = 4
PERIOD = 512


def _sc_gather(ids_flat, table):
    n = ids_flat.shape[0]
    d = table.shape[1]
    nch = n // (NW * CHUNK)
    ngrp = nch // GROUP
    rows_per_w = nch * CHUNK
    ids3 = ids_flat.reshape(NW, nch, CHUNK)
    mesh = plsc.VectorSubcoreMesh(core_axis_name="c", subcore_axis_name="s")

    @functools.partial(
        pl.kernel,
        out_type=jax.ShapeDtypeStruct((n, d), table.dtype),
        mesh=mesh,
        scratch_types=[
            pltpu.VMEM((nch, CHUNK), jnp.int32),
            pltpu.VMEM((GROUP * CHUNK, d), table.dtype),
            pltpu.SemaphoreType.DMA,
        ],
        compiler_params=pltpu.CompilerParams(use_tc_tiling_on_sc=False),
    )
    def gather_kernel(ids_hbm, table_hbm, out_hbm, idx_v, rows_v, sem):
        wid = lax.axis_index("s") * NC + lax.axis_index("c")
        base = wid * rows_per_w
        pltpu.sync_copy(ids_hbm.at[wid], idx_v)

        @pl.loop(0, ngrp)
        def _grp(g):
            waits = []
            for k in range(GROUP):
                c = g * GROUP + k
                waits.append(pltpu.async_copy(
                    table_hbm.at[idx_v.at[c]],
                    rows_v.at[pl.ds(k * CHUNK, CHUNK)],
                    sem))
            for w in waits:
                w.wait()
            pltpu.sync_copy(
                rows_v,
                out_hbm.at[pl.ds(base + g * (GROUP * CHUNK), GROUP * CHUNK)])

    return gather_kernel(ids3, table)


def _tc_encode_t(emb2, ego_p, wblk, w2, b2, bz):
    n2, dd = emb2.shape
    d = dd // 2
    hb = bz // 2
    sl = n2 // hb
    reps = bz // PERIOD

    def body(emb_ref, ego_ref, wb_ref, w2_ref, b_ref, out_ref):
        base = lax.dot_general(
            w2_ref[...], ego_ref[0],
            (((0,), (1,)), ((), ())),
            preferred_element_type=jnp.float32,
            precision=lax.Precision.HIGHEST) + b_ref[...]
        addend = jnp.concatenate([base] * reps, axis=1)

        h2 = lax.dot_general(
            wb_ref[...].astype(jnp.bfloat16),
            emb_ref[...].astype(jnp.bfloat16),
            (((0,), (1,)), ((), ())),
            preferred_element_type=jnp.float32)
        slab = jnp.concatenate([h2[:d, :], h2[d:, :]], axis=1)
        out_ref[...] = jnp.maximum(slab + addend, 0.0)

    return pl.pallas_call(
        body,
        grid=(sl,),
        in_specs=[
            pl.BlockSpec((hb, dd), lambda s: (s, 0)),
            pl.BlockSpec((1, PERIOD, 3), lambda s: (s, 0, 0)),
            pl.BlockSpec(wblk.shape, lambda s: (0, 0)),
            pl.BlockSpec(w2.shape, lambda s: (0, 0)),
            pl.BlockSpec(b2.shape, lambda s: (0, 0)),
        ],
        out_specs=pl.BlockSpec((d, bz), lambda s: (s, 0)),
        out_shape=jax.ShapeDtypeStruct((sl * d, bz), jnp.float32),
    )(emb2, ego_p, wblk, w2, b2)


def kernel(input_ids, ego_info, table, W, b):
    bz, sl = input_ids.shape
    d = table.shape[1]
    hb = bz // 2
    ids_t = input_ids.T.astype(jnp.int32)
    ids_flat = ids_t.reshape(sl, 2, hb).transpose(0, 2, 1).reshape(bz * sl)
    emb = _sc_gather(ids_flat, table)
    emb2 = emb.reshape(bz * sl // 2, 2 * d)
    w1 = W[:d]
    w2 = W[d:]
    b2 = b.reshape(d, 1)
    wblk = (jnp.zeros((2 * d, 2 * d), jnp.float32)
            .at[:d, :d].set(w1).at[d:, d:].set(w1))
    reps = (PERIOD * sl) // bz
    ego_p = jnp.tile(ego_info, (reps, 1)).reshape(PERIOD, sl, 3)
    ego_p = ego_p.transpose(1, 0, 2)
    out_t = _tc_encode_t(emb2, ego_p, wblk, w2, b2, bz)
    return out_t.reshape(sl, d, bz).transpose(2, 0, 1)


# speedup vs baseline: 2.2585x; 1.1559x over previous
import functools

import jax
import jax.numpy as jnp
from jax import lax
from jax.experimental import pallas as pl
from jax.experimental.pallas import tpu as pltpu
from jax.experimental.pallas import tpu_sc as plsc

NC = 2
NS = 16
NW = NC * NS
CHUNK = 128
GROUP = 4
PERIOD = 512


def _sc_gather2(ids_a, ids_b, table):
    half = ids_a.shape[0]
    n = 2 * half
    d = table.shape[1]
    nch = n // (NW * CHUNK)
    ngrp = nch // GROUP
    rows_per_w = nch * CHUNK
    hw = half // NW
    ids_a2 = ids_a.reshape(NW, hw)
    ids_b2 = ids_b.reshape(NW, hw)
    mesh = plsc.VectorSubcoreMesh(core_axis_name="c", subcore_axis_name="s")

    @functools.partial(
        pl.kernel,
        out_type=jax.ShapeDtypeStruct((n, d), table.dtype),
        mesh=mesh,
        scratch_types=[
            pltpu.VMEM((hw,), jnp.int32),
            pltpu.VMEM((hw,), jnp.int32),
            pltpu.VMEM((2 * hw,), jnp.int32),
            pltpu.VMEM((GROUP * CHUNK, d), table.dtype),
            pltpu.SemaphoreType.DMA,
        ],
        compiler_params=pltpu.CompilerParams(
            use_tc_tiling_on_sc=False, needs_layout_passes=False),
    )
    def gather_kernel(a_hbm, b_hbm, table_hbm, out_hbm,
                      a_v, b_v, idx_v, rows_v, sem):
        wid = lax.axis_index("s") * NC + lax.axis_index("c")
        base = wid * rows_per_w
        pltpu.sync_copy(a_hbm.at[wid], a_v)
        pltpu.sync_copy(b_hbm.at[wid], b_v)

        @pl.loop(0, hw // 16)
        def _ilv(g):
            pos = g * 32 + 2 * lax.iota(jnp.int32, 16)
            plsc.store_scatter(idx_v, [pos], a_v[pl.ds(g * 16, 16)])
            plsc.store_scatter(idx_v, [pos + 1], b_v[pl.ds(g * 16, 16)])

        @pl.loop(0, ngrp)
        def _grp(g):
            waits = []
            for k in range(GROUP):
                c = g * GROUP + k
                waits.append(pltpu.async_copy(
                    table_hbm.at[idx_v.at[pl.ds(c * CHUNK, CHUNK)]],
                    rows_v.at[pl.ds(k * CHUNK, CHUNK)],
                    sem))
            for w in waits:
                w.wait()
            pltpu.sync_copy(
                rows_v,
                out_hbm.at[pl.ds(base + g * (GROUP * CHUNK), GROUP * CHUNK)])

    return gather_kernel(ids_a2, ids_b2, table)


def _tc_encode_t(emb2, ego_p, wblk, w2, b2, bz):
    n2, dd = emb2.shape
    d = dd // 2
    hb = bz // 2
    sl = n2 // hb
    reps = bz // PERIOD

    def body(emb_ref, ego_ref, wb_ref, w2_ref, b_ref, out_ref):
        base = lax.dot_general(
            w2_ref[...], ego_ref[0],
            (((0,), (1,)), ((), ())),
            preferred_element_type=jnp.float32,
            precision=lax.Precision.HIGHEST) + b_ref[...]
        addend = jnp.concatenate([base] * reps, axis=1)

        h2 = lax.dot_general(
            wb_ref[...].astype(jnp.bfloat16),
            emb_ref[...].astype(jnp.bfloat16),
            (((0,), (1,)), ((), ())),
            preferred_element_type=jnp.float32)
        slab = jnp.concatenate([h2[:d, :], h2[d:, :]], axis=1)
        out_ref[...] = jnp.maximum(slab + addend, 0.0)

    return pl.pallas_call(
        body,
        grid=(sl,),
        in_specs=[
            pl.BlockSpec((hb, dd), lambda s: (s, 0)),
            pl.BlockSpec((1, PERIOD, 3), lambda s: (s, 0, 0)),
            pl.BlockSpec(wblk.shape, lambda s: (0, 0)),
            pl.BlockSpec(w2.shape, lambda s: (0, 0)),
            pl.BlockSpec(b2.shape, lambda s: (0, 0)),
        ],
        out_specs=pl.BlockSpec((d, bz), lambda s: (s, 0)),
        out_shape=jax.ShapeDtypeStruct((sl * d, bz), jnp.float32),
    )(emb2, ego_p, wblk, w2, b2)


def kernel(input_ids, ego_info, table, W, b):
    bz, sl = input_ids.shape
    d = table.shape[1]
    hb = bz // 2
    ids_t = input_ids.T.astype(jnp.int32)
    ids_a = ids_t[:, :hb].reshape(sl * hb)
    ids_b = ids_t[:, hb:].reshape(sl * hb)
    emb = _sc_gather2(ids_a, ids_b, table)
    emb2 = emb.reshape(bz * sl // 2, 2 * d)
    w1 = W[:d]
    w2 = W[d:]
    b2 = b.reshape(d, 1)
    wblk = (jnp.zeros((2 * d, 2 * d), jnp.float32)
            .at[:d, :d].set(w1).at[d:, d:].set(w1))
    reps = (PERIOD * sl) // bz
    ego_p = jnp.tile(ego_info, (reps, 1)).reshape(PERIOD, sl, 3)
    ego_p = ego_p.transpose(1, 0, 2)
    out_t = _tc_encode_t(emb2, ego_p, wblk, w2, b2, bz)
    return out_t.reshape(sl, d, bz).transpose(2, 0, 1)


# speedup vs baseline: 3.0085x; 1.3321x over previous
import functools

import jax
import jax.numpy as jnp
from jax import lax
from jax.experimental import pallas as pl
from jax.experimental.pallas import tpu as pltpu
from jax.experimental.pallas import tpu_sc as plsc

NC = 2
NS = 16
NW = NC * NS
CHUNK = 128
GROUP = 4
PERIOD = 512


NB = 4096


def _tc_repack(table_t):
    d, v = table_t.shape
    grid = (v + NB - 1) // NB
    h = NB // 2

    def body(in_ref, out_ref):
        xt = in_ref[...].T
        out_ref[...] = jnp.concatenate([xt[:h], xt[h:]], axis=1)

    return pl.pallas_call(
        body,
        grid=(grid,),
        in_specs=[pl.BlockSpec((d, NB), lambda i: (0, i))],
        out_specs=pl.BlockSpec((h, 2 * d), lambda i: (i, 0)),
        out_shape=jax.ShapeDtypeStruct((grid * h, 2 * d), jnp.float32),
    )(table_t)


def _perm_ids(ids):
    h = NB // 2
    g = ids // NB
    j = ids % NB
    return 2 * (g * h + (j % h)) + (j // h)


def _sc_gather2(ids_a, ids_b, table):
    half = ids_a.shape[0]
    n = 2 * half
    d = table.shape[1]
    nch = n // (NW * CHUNK)
    ngrp = nch // GROUP
    rows_per_w = nch * CHUNK
    hw = half // NW
    ids_a2 = ids_a.reshape(NW, hw)
    ids_b2 = ids_b.reshape(NW, hw)
    mesh = plsc.VectorSubcoreMesh(core_axis_name="c", subcore_axis_name="s")

    @functools.partial(
        pl.kernel,
        out_type=jax.ShapeDtypeStruct((n, d), table.dtype),
        mesh=mesh,
        scratch_types=[
            pltpu.VMEM((hw,), jnp.int32),
            pltpu.VMEM((hw,), jnp.int32),
            pltpu.VMEM((2 * hw,), jnp.int32),
            pltpu.VMEM((GROUP * CHUNK, d), table.dtype),
            pltpu.SemaphoreType.DMA,
        ],
        compiler_params=pltpu.CompilerParams(
            use_tc_tiling_on_sc=False, needs_layout_passes=False),
    )
    def gather_kernel(a_hbm, b_hbm, table_hbm, out_hbm,
                      a_v, b_v, idx_v, rows_v, sem):
        wid = lax.axis_index("s") * NC + lax.axis_index("c")
        base = wid * rows_per_w
        pltpu.sync_copy(a_hbm.at[wid], a_v)
        pltpu.sync_copy(b_hbm.at[wid], b_v)

        @pl.loop(0, hw // 16)
        def _ilv(g):
            pos = g * 32 + 2 * lax.iota(jnp.int32, 16)
            plsc.store_scatter(idx_v, [pos], a_v[pl.ds(g * 16, 16)])
            plsc.store_scatter(idx_v, [pos + 1], b_v[pl.ds(g * 16, 16)])

        @pl.loop(0, ngrp)
        def _grp(g):
            waits = []
            for k in range(GROUP):
                c = g * GROUP + k
                waits.append(pltpu.async_copy(
                    table_hbm.at[idx_v.at[pl.ds(c * CHUNK, CHUNK)]],
                    rows_v.at[pl.ds(k * CHUNK, CHUNK)],
                    sem))
            for w in waits:
                w.wait()
            pltpu.sync_copy(
                rows_v,
                out_hbm.at[pl.ds(base + g * (GROUP * CHUNK), GROUP * CHUNK)])

    return gather_kernel(ids_a2, ids_b2, table)


def _tc_encode_t(emb2, ego_p, wblk, w2, b2, bz):
    n2, dd = emb2.shape
    d = dd // 2
    hb = bz // 2
    sl = n2 // hb
    reps = bz // PERIOD

    def body(emb_ref, ego_ref, wb_ref, w2_ref, b_ref, out_ref):
        base = lax.dot_general(
            w2_ref[...], ego_ref[0],
            (((0,), (1,)), ((), ())),
            preferred_element_type=jnp.float32,
            precision=lax.Precision.HIGHEST) + b_ref[...]
        addend = jnp.concatenate([base] * reps, axis=1)

        h2 = lax.dot_general(
            wb_ref[...].astype(jnp.bfloat16),
            emb_ref[...].astype(jnp.bfloat16),
            (((0,), (1,)), ((), ())),
            preferred_element_type=jnp.float32)
        slab = jnp.concatenate([h2[:d, :], h2[d:, :]], axis=1)
        out_ref[...] = jnp.maximum(slab + addend, 0.0)

    return pl.pallas_call(
        body,
        grid=(sl,),
        in_specs=[
            pl.BlockSpec((hb, dd), lambda s: (s, 0)),
            pl.BlockSpec((1, PERIOD, 3), lambda s: (s, 0, 0)),
            pl.BlockSpec(wblk.shape, lambda s: (0, 0)),
            pl.BlockSpec(w2.shape, lambda s: (0, 0)),
            pl.BlockSpec(b2.shape, lambda s: (0, 0)),
        ],
        out_specs=pl.BlockSpec((d, bz), lambda s: (s, 0)),
        out_shape=jax.ShapeDtypeStruct((sl * d, bz), jnp.float32),
    )(emb2, ego_p, wblk, w2, b2)


def kernel(input_ids, ego_info, table, W, b):
    bz, sl = input_ids.shape
    d = table.shape[1]
    hb = bz // 2
    ids_t = input_ids.T.astype(jnp.int32)
    ids_a = _perm_ids(ids_t[:, :hb]).reshape(sl * hb)
    ids_b = _perm_ids(ids_t[:, hb:]).reshape(sl * hb)
    table_r = _tc_repack(table.T)
    table_rm = table_r.reshape(2 * table_r.shape[0], d)
    emb = _sc_gather2(ids_a, ids_b, table_rm)
    emb2 = emb.reshape(bz * sl // 2, 2 * d)
    w1 = W[:d]
    w2 = W[d:]
    b2 = b.reshape(d, 1)
    wblk = (jnp.zeros((2 * d, 2 * d), jnp.float32)
            .at[:d, :d].set(w1).at[d:, d:].set(w1))
    reps = (PERIOD * sl) // bz
    ego_p = jnp.tile(ego_info, (reps, 1)).reshape(PERIOD, sl, 3)
    ego_p = ego_p.transpose(1, 0, 2)
    out_t = _tc_encode_t(emb2, ego_p, wblk, w2, b2, bz)
    return out_t.reshape(sl, d, bz).transpose(2, 0, 1)


# speedup vs baseline: 3.6098x; 1.1999x over previous
import functools

import jax
import jax.numpy as jnp
from jax import lax
from jax.experimental import pallas as pl
from jax.experimental.pallas import tpu as pltpu
from jax.experimental.pallas import tpu_sc as plsc

NC = 2
NS = 16
NW = NC * NS
CHUNK = 128
GROUP = 4
PERIOD = 512


NB = 8192


def _tc_repack(table_t):
    d, v = table_t.shape
    grid = (v + NB - 1) // NB
    h = NB // 2

    def body(in_ref, out_ref):
        xt = in_ref[...].T
        out_ref[...] = jnp.concatenate([xt[:h], xt[h:]], axis=1)

    return pl.pallas_call(
        body,
        grid=(grid,),
        in_specs=[pl.BlockSpec((d, NB), lambda i: (0, i))],
        out_specs=pl.BlockSpec((h, 2 * d), lambda i: (i, 0)),
        out_shape=jax.ShapeDtypeStruct((grid * h, 2 * d), jnp.float32),
    )(table_t)


def _perm_ids(ids):
    h = NB // 2
    g = ids // NB
    j = ids % NB
    return 2 * (g * h + (j % h)) + (j // h)


def _sc_gather2(ids_a, ids_b, table):
    half = ids_a.shape[0]
    n = 2 * half
    d = table.shape[1]
    nch = n // (NW * CHUNK)
    ngrp = nch // GROUP
    rows_per_w = nch * CHUNK
    hw = half // NW
    ids_a2 = ids_a.reshape(NW, hw)
    ids_b2 = ids_b.reshape(NW, hw)
    mesh = plsc.VectorSubcoreMesh(core_axis_name="c", subcore_axis_name="s")

    @functools.partial(
        pl.kernel,
        out_type=jax.ShapeDtypeStruct((n, d), table.dtype),
        mesh=mesh,
        scratch_types=[
            pltpu.VMEM((hw,), jnp.int32),
            pltpu.VMEM((hw,), jnp.int32),
            pltpu.VMEM((2 * hw,), jnp.int32),
            pltpu.VMEM((GROUP * CHUNK, d), table.dtype),
            pltpu.SemaphoreType.DMA,
        ],
        compiler_params=pltpu.CompilerParams(
            use_tc_tiling_on_sc=False, needs_layout_passes=False),
    )
    def gather_kernel(a_hbm, b_hbm, table_hbm, out_hbm,
                      a_v, b_v, idx_v, rows_v, sem):
        wid = lax.axis_index("s") * NC + lax.axis_index("c")
        base = wid * rows_per_w
        pltpu.sync_copy(a_hbm.at[wid], a_v)
        pltpu.sync_copy(b_hbm.at[wid], b_v)

        @pl.loop(0, hw // 16)
        def _ilv(g):
            pos = g * 32 + 2 * lax.iota(jnp.int32, 16)
            plsc.store_scatter(idx_v, [pos], a_v[pl.ds(g * 16, 16)])
            plsc.store_scatter(idx_v, [pos + 1], b_v[pl.ds(g * 16, 16)])

        @pl.loop(0, ngrp)
        def _grp(g):
            waits = []
            for k in range(GROUP):
                c = g * GROUP + k
                waits.append(pltpu.async_copy(
                    table_hbm.at[idx_v.at[pl.ds(c * CHUNK, CHUNK)]],
                    rows_v.at[pl.ds(k * CHUNK, CHUNK)],
                    sem))
            for w in waits:
                w.wait()
            pltpu.sync_copy(
                rows_v,
                out_hbm.at[pl.ds(base + g * (GROUP * CHUNK), GROUP * CHUNK)])

    return gather_kernel(ids_a2, ids_b2, table)


def _tc_encode_t(emb2, ego_p, wblk, w2, b2, bz):
    n2, dd = emb2.shape
    d = dd // 2
    hb = bz // 2
    sl = n2 // hb
    reps = bz // PERIOD

    def body(emb_ref, ego_ref, wb_ref, w2_ref, b_ref, out_ref):
        h2 = lax.dot_general(
            wb_ref[...].astype(jnp.bfloat16),
            emb_ref[...].astype(jnp.bfloat16),
            (((0,), (1,)), ((), ())),
            preferred_element_type=jnp.float32)
        slabs = []
        for k in range(2):
            base = lax.dot_general(
                w2_ref[...], ego_ref[k],
                (((0,), (1,)), ((), ())),
                preferred_element_type=jnp.float32,
                precision=lax.Precision.HIGHEST) + b_ref[...]
            addend = jnp.concatenate([base] * reps, axis=1)
            cols = h2[:, k * hb:(k + 1) * hb]
            slab = jnp.concatenate([cols[:d, :], cols[d:, :]], axis=1)
            slabs.append(jnp.maximum(slab + addend, 0.0))
        out_ref[...] = jnp.concatenate(slabs, axis=0)

    return pl.pallas_call(
        body,
        grid=(sl // 2,),
        in_specs=[
            pl.BlockSpec((bz, dd), lambda g: (g, 0)),
            pl.BlockSpec((2, PERIOD, 3), lambda g: (g, 0, 0)),
            pl.BlockSpec(wblk.shape, lambda g: (0, 0)),
            pl.BlockSpec(w2.shape, lambda g: (0, 0)),
            pl.BlockSpec(b2.shape, lambda g: (0, 0)),
        ],
        out_specs=pl.BlockSpec((2 * d, bz), lambda g: (g, 0)),
        out_shape=jax.ShapeDtypeStruct((sl * d, bz), jnp.float32),
    )(emb2, ego_p, wblk, w2, b2)


def kernel(input_ids, ego_info, table, W, b):
    bz, sl = input_ids.shape
    d = table.shape[1]
    hb = bz // 2
    ids_t = input_ids.T.astype(jnp.int32)
    ids_a = _perm_ids(ids_t[:, :hb]).reshape(sl * hb)
    ids_b = _perm_ids(ids_t[:, hb:]).reshape(sl * hb)
    table_r = _tc_repack(table.T)
    table_rm = table_r.reshape(2 * table_r.shape[0], d)
    emb = _sc_gather2(ids_a, ids_b, table_rm)
    emb2 = emb.reshape(bz * sl // 2, 2 * d)
    w1 = W[:d]
    w2 = W[d:]
    b2 = b.reshape(d, 1)
    wblk = (jnp.zeros((2 * d, 2 * d), jnp.float32)
            .at[:d, :d].set(w1).at[d:, d:].set(w1))
    reps = (PERIOD * sl) // bz
    ego_p = jnp.tile(ego_info, (reps, 1)).reshape(PERIOD, sl, 3)
    ego_p = ego_p.transpose(1, 0, 2)
    out_t = _tc_encode_t(emb2, ego_p, wblk, w2, b2, bz)
    return out_t.reshape(sl, d, bz).transpose(2, 0, 1)


# speedup vs baseline: 3.8755x; 1.0736x over previous
import functools

import jax
import jax.numpy as jnp
from jax import lax
from jax.experimental import pallas as pl
from jax.experimental.pallas import tpu as pltpu
from jax.experimental.pallas import tpu_sc as plsc

NC = 2
NS = 16
NW = NC * NS
CHUNK = 128
GROUP = 8
PERIOD = 512


NB = 16384


def _tc_repack(table_t):
    d, v = table_t.shape
    grid = (v + NB - 1) // NB
    h = NB // 2

    def body(in_ref, out_ref):
        xt = in_ref[...].T
        out_ref[...] = jnp.concatenate([xt[:h], xt[h:]], axis=1)

    return pl.pallas_call(
        body,
        grid=(grid,),
        in_specs=[pl.BlockSpec((d, NB), lambda i: (0, i))],
        out_specs=pl.BlockSpec((h, 2 * d), lambda i: (i, 0)),
        out_shape=jax.ShapeDtypeStruct((grid * h, 2 * d), jnp.float32),
    )(table_t)


def _perm_ids(ids):
    h = NB // 2
    g = ids // NB
    j = ids % NB
    return 2 * (g * h + (j % h)) + (j // h)


def _sc_gather2(ids_a, ids_b, table):
    half = ids_a.shape[0]
    n = 2 * half
    d = table.shape[1]
    nch = n // (NW * CHUNK)
    ngrp = nch // GROUP
    rows_per_w = nch * CHUNK
    hw = half // NW
    ids_a2 = ids_a.reshape(NW, hw)
    ids_b2 = ids_b.reshape(NW, hw)
    mesh = plsc.VectorSubcoreMesh(core_axis_name="c", subcore_axis_name="s")

    @functools.partial(
        pl.kernel,
        out_type=jax.ShapeDtypeStruct((n, d), table.dtype),
        mesh=mesh,
        scratch_types=[
            pltpu.VMEM((hw,), jnp.int32),
            pltpu.VMEM((hw,), jnp.int32),
            pltpu.VMEM((2 * hw,), jnp.int32),
            pltpu.VMEM((GROUP * CHUNK, d), table.dtype),
            pltpu.SemaphoreType.DMA,
        ],
        compiler_params=pltpu.CompilerParams(
            use_tc_tiling_on_sc=False, needs_layout_passes=False),
    )
    def gather_kernel(a_hbm, b_hbm, table_hbm, out_hbm,
                      a_v, b_v, idx_v, rows_v, sem):
        wid = lax.axis_index("s") * NC + lax.axis_index("c")
        base = wid * rows_per_w
        pltpu.sync_copy(a_hbm.at[wid], a_v)
        pltpu.sync_copy(b_hbm.at[wid], b_v)

        @pl.loop(0, hw // 16)
        def _ilv(g):
            pos = g * 32 + 2 * lax.iota(jnp.int32, 16)
            plsc.store_scatter(idx_v, [pos], a_v[pl.ds(g * 16, 16)])
            plsc.store_scatter(idx_v, [pos + 1], b_v[pl.ds(g * 16, 16)])

        @pl.loop(0, ngrp)
        def _grp(g):
            waits = []
            for k in range(GROUP):
                c = g * GROUP + k
                waits.append(pltpu.async_copy(
                    table_hbm.at[idx_v.at[pl.ds(c * CHUNK, CHUNK)]],
                    rows_v.at[pl.ds(k * CHUNK, CHUNK)],
                    sem))
            for w in waits:
                w.wait()
            pltpu.sync_copy(
                rows_v,
                out_hbm.at[pl.ds(base + g * (GROUP * CHUNK), GROUP * CHUNK)])

    return gather_kernel(ids_a2, ids_b2, table)


def _tc_encode_t(emb2, ego_p, wblk, w2, b2, bz):
    n2, dd = emb2.shape
    d = dd // 2
    hb = bz // 2
    sl = n2 // hb
    reps = bz // PERIOD

    def body(emb_ref, ego_ref, wb_ref, w2_ref, b_ref, out_ref):
        h2 = lax.dot_general(
            wb_ref[...].astype(jnp.bfloat16),
            emb_ref[...].astype(jnp.bfloat16),
            (((0,), (1,)), ((), ())),
            preferred_element_type=jnp.float32)
        slabs = []
        for k in range(2):
            base = lax.dot_general(
                w2_ref[...], ego_ref[k],
                (((0,), (1,)), ((), ())),
                preferred_element_type=jnp.float32,
                precision=lax.Precision.HIGHEST) + b_ref[...]
            addend = jnp.concatenate([base] * reps, axis=1)
            cols = h2[:, k * hb:(k + 1) * hb]
            slab = jnp.concatenate([cols[:d, :], cols[d:, :]], axis=1)
            slabs.append(jnp.maximum(slab + addend, 0.0))
        out_ref[...] = jnp.concatenate(slabs, axis=0)

    return pl.pallas_call(
        body,
        grid=(sl // 2,),
        in_specs=[
            pl.BlockSpec((bz, dd), lambda g: (g, 0)),
            pl.BlockSpec((2, PERIOD, 3), lambda g: (g, 0, 0)),
            pl.BlockSpec(wblk.shape, lambda g: (0, 0)),
            pl.BlockSpec(w2.shape, lambda g: (0, 0)),
            pl.BlockSpec(b2.shape, lambda g: (0, 0)),
        ],
        out_specs=pl.BlockSpec((2 * d, bz), lambda g: (g, 0)),
        out_shape=jax.ShapeDtypeStruct((sl * d, bz), jnp.float32),
    )(emb2, ego_p, wblk, w2, b2)


def kernel(input_ids, ego_info, table, W, b):
    bz, sl = input_ids.shape
    d = table.shape[1]
    hb = bz // 2
    ids_t = input_ids.T.astype(jnp.int32)
    ids_a = _perm_ids(ids_t[:, :hb]).reshape(sl * hb)
    ids_b = _perm_ids(ids_t[:, hb:]).reshape(sl * hb)
    table_r = _tc_repack(table.T)
    table_rm = table_r.reshape(2 * table_r.shape[0], d)
    emb = _sc_gather2(ids_a, ids_b, table_rm)
    emb2 = emb.reshape(bz * sl // 2, 2 * d)
    w1 = W[:d]
    w2 = W[d:]
    b2 = b.reshape(d, 1)
    wblk = (jnp.zeros((2 * d, 2 * d), jnp.float32)
            .at[:d, :d].set(w1).at[d:, d:].set(w1))
    reps = (PERIOD * sl) // bz
    ego_p = jnp.tile(ego_info, (reps, 1)).reshape(PERIOD, sl, 3)
    ego_p = ego_p.transpose(1, 0, 2)
    out_t = _tc_encode_t(emb2, ego_p, wblk, w2, b2, bz)
    return out_t.reshape(sl, d, bz).transpose(2, 0, 1)
